# baseline (speedup 1.0000x reference)
import functools

import jax
import jax.numpy as jnp
from jax.experimental import pallas as pl

K_KEEP = 21


def _sim_topk_kernel(hs_ref, hall_ref, out_ref, *, k_keep):
    out_ref[...] = jax.lax.dot_general(
        hs_ref[...], hall_ref[...],
        (((1,), (1,)), ((), ())),
        preferred_element_type=jnp.float32,
    )
    s = out_ref[...]
    n_cols = s.shape[1]
    neg = jnp.float32(-jnp.inf)

    cw = 128
    ws = [s[:, c:min(c + cw, n_cols)] for c in range(0, n_cols, cw)]
    tops = [[jnp.max(w, axis=1, keepdims=True) for w in ws]]
    for r in range(3):
        tops.append([
            jnp.max(jnp.where(w < m, w, neg), axis=1, keepdims=True)
            for w, m in zip(ws, tops[-1])
        ])
    p = jnp.concatenate([m for ms in tops for m in ms], axis=1)

    t = jnp.max(p, axis=1, keepdims=True)
    for _ in range(k_keep - 1):
        t = jnp.max(jnp.where(p < t, p, neg), axis=1, keepdims=True)

    km1 = jnp.float32(k_keep - 1)
    kk = jnp.float32(k_keep)
    sge = s >= t
    out_ref[...] = jnp.where(sge, jnp.maximum(s, 0.0), 0.0)
    cge = jnp.sum(sge.astype(jnp.float32), axis=1, keepdims=True)
    suspicious = cge > kk

    def _repair(t0):
        def _cond(carry):
            _, flag = carry
            return jnp.any(flag > 0)

        def _body(carry):
            u, _ = carry
            cgt = jnp.sum((s > u).astype(jnp.float32), axis=1,
                          keepdims=True)
            needs = cgt > km1
            mn = jnp.min(jnp.where(s > u, s, jnp.float32(jnp.inf)),
                         axis=1, keepdims=True)
            u = jnp.where(needs, mn, u)
            return u, jnp.where(needs, 1.0, 0.0)

        u, _ = jax.lax.while_loop(
            _cond, _body, (t0, jnp.where(suspicious, 1.0, 0.0)))
        out_ref[...] = jnp.where(s >= u, jnp.maximum(s, 0.0), 0.0)

    jax.lax.cond(jnp.any(suspicious), _repair, lambda _: None, t)


@functools.partial(jax.jit, static_argnames=("interpret",))
def kernel(x, w0, w1, interpret=False):
    n, d = x.shape
    rs = 200 if n % 200 == 0 else n

    h = jax.nn.relu(x * w0) * w1
    norm = jnp.linalg.norm(h, axis=-1, keepdims=True)
    h = (h / jnp.clip(norm, 1e-12, None)).astype(jnp.bfloat16)

    out = pl.pallas_call(
        functools.partial(_sim_topk_kernel, k_keep=K_KEEP),
        grid=(n // rs,),
        in_specs=[
            pl.BlockSpec((rs, d), lambda i: (i, 0)),
            pl.BlockSpec((n, d), lambda i: (0, 0)),
        ],
        out_specs=pl.BlockSpec((rs, n), lambda i: (i, 0)),
        out_shape=jax.ShapeDtypeStruct((n, n), jnp.float32),
        interpret=interpret,
    )(h, h)
    return out


# speedup vs baseline: 1.0348x; 1.0348x over previous
import functools

import jax
import jax.numpy as jnp
from jax.experimental import pallas as pl

K_KEEP = 21


def _sim_topk_kernel(hs_ref, hall_ref, out_ref, *, k_keep):
    out_ref[...] = jax.lax.dot_general(
        hs_ref[...], hall_ref[...],
        (((1,), (1,)), ((), ())),
        preferred_element_type=jnp.float32,
    )
    s = out_ref[...]
    n_cols = s.shape[1]
    neg = jnp.float32(-jnp.inf)

    cw = 128
    ws = [s[:, c:min(c + cw, n_cols)] for c in range(0, n_cols, cw)]
    tops = []
    for r in range(4):
        ms = [jnp.max(w, axis=1, keepdims=True) for w in ws]
        tops.extend(ms)
        if r < 3:
            ws = [jnp.where(w >= m, neg, w) for w, m in zip(ws, ms)]
    p = jnp.concatenate(tops, axis=1)

    t = None
    for _ in range(k_keep):
        t = jnp.max(p, axis=1, keepdims=True)
        p = jnp.where(p >= t, neg, p)

    km1 = jnp.float32(k_keep - 1)
    cnt = jnp.sum((s > t).astype(jnp.float32), axis=1, keepdims=True)

    def _cond(carry):
        _, c = carry
        return jnp.any(c > km1)

    def _body(carry):
        u, c = carry
        mn = jnp.min(jnp.where(s > u, s, jnp.float32(jnp.inf)),
                     axis=1, keepdims=True)
        u = jnp.where(c > km1, mn, u)
        c = jnp.sum((s > u).astype(jnp.float32), axis=1, keepdims=True)
        return u, c

    t, _ = jax.lax.while_loop(_cond, _body, (t, cnt))
    out_ref[...] = jnp.where(s >= t, jnp.maximum(s, 0.0), 0.0)


@functools.partial(jax.jit, static_argnames=("interpret",))
def kernel(x, w0, w1, interpret=False):
    n, d = x.shape
    rs = 200 if n % 200 == 0 else n

    h = jax.nn.relu(x * w0) * w1
    norm = jnp.linalg.norm(h, axis=-1, keepdims=True)
    h = (h / jnp.clip(norm, 1e-12, None)).astype(jnp.bfloat16)

    out = pl.pallas_call(
        functools.partial(_sim_topk_kernel, k_keep=K_KEEP),
        grid=(n // rs,),
        in_specs=[
            pl.BlockSpec((rs, d), lambda i: (i, 0)),
            pl.BlockSpec((n, d), lambda i: (0, 0)),
        ],
        out_specs=pl.BlockSpec((rs, n), lambda i: (i, 0)),
        out_shape=jax.ShapeDtypeStruct((n, n), jnp.float32),
        interpret=interpret,
    )(h, h)
    return out


# speedup vs baseline: 1.3079x; 1.2639x over previous
import functools

import jax
import jax.numpy as jnp
from jax.experimental import pallas as pl

K_KEEP = 21


def _sim_topk_kernel(hs_ref, hall_ref, out_ref, *, k_keep):
    out_ref[...] = jax.lax.dot_general(
        hs_ref[...], hall_ref[...],
        (((1,), (1,)), ((), ())),
        preferred_element_type=jnp.float32,
    )
    s = out_ref[...]
    n_cols = s.shape[1]
    neg = jnp.float32(-jnp.inf)

    cw = 256
    nrounds = 5
    ws = [s[:, c:min(c + cw, n_cols)] for c in range(0, n_cols, cw)]
    tops = []
    for r in range(nrounds):
        ms = [jnp.max(w, axis=1, keepdims=True) for w in ws]
        tops.extend(ms)
        if r < nrounds - 1:
            ws = [jnp.where(w >= m, neg, w) for w, m in zip(ws, ms)]
    p = jnp.concatenate(tops, axis=1)

    t = None
    for _ in range(k_keep):
        t = jnp.max(p, axis=1, keepdims=True)
        p = jnp.where(p >= t, neg, p)

    km1 = jnp.float32(k_keep - 1)
    cnt = jnp.sum((s > t).astype(jnp.float32), axis=1, keepdims=True)

    def _cond(carry):
        _, c = carry
        return jnp.any(c > km1)

    def _body(carry):
        u, c = carry
        mn = jnp.min(jnp.where(s > u, s, jnp.float32(jnp.inf)),
                     axis=1, keepdims=True)
        u = jnp.where(c > km1, mn, u)
        c = jnp.sum((s > u).astype(jnp.float32), axis=1, keepdims=True)
        return u, c

    t, _ = jax.lax.while_loop(_cond, _body, (t, cnt))
    out_ref[...] = jnp.where(s >= t, jnp.maximum(s, 0.0), 0.0)


@functools.partial(jax.jit, static_argnames=("interpret",))
def kernel(x, w0, w1, interpret=False):
    n, d = x.shape
    rs = 200 if n % 200 == 0 else n

    h = jax.nn.relu(x * w0) * w1
    norm = jnp.linalg.norm(h, axis=-1, keepdims=True)
    h = (h / jnp.clip(norm, 1e-12, None)).astype(jnp.bfloat16)

    out = pl.pallas_call(
        functools.partial(_sim_topk_kernel, k_keep=K_KEEP),
        grid=(n // rs,),
        in_specs=[
            pl.BlockSpec((rs, d), lambda i: (i, 0)),
            pl.BlockSpec((n, d), lambda i: (0, 0)),
        ],
        out_specs=pl.BlockSpec((rs, n), lambda i: (i, 0)),
        out_shape=jax.ShapeDtypeStruct((n, n), jnp.float32),
        interpret=interpret,
    )(h, h)
    return out


# speedup vs baseline: 1.3194x; 1.0088x over previous
import functools

import jax
import jax.numpy as jnp
from jax.experimental import pallas as pl

K_KEEP = 21


def _sim_topk_kernel(hs_ref, hall_ref, out_ref, *, k_keep):
    s = jax.lax.dot_general(
        hs_ref[...], hall_ref[...],
        (((1,), (1,)), ((), ())),
        preferred_element_type=jnp.float32,
    )
    n_cols = s.shape[1]
    neg = jnp.float32(-jnp.inf)

    cw = 256
    nrounds = 5
    ws = [s[:, c:min(c + cw, n_cols)] for c in range(0, n_cols, cw)]
    tops = []
    for r in range(nrounds):
        ms = [jnp.max(w, axis=1, keepdims=True) for w in ws]
        tops.extend(ms)
        if r < nrounds - 1:
            ws = [jnp.where(w >= m, neg, w) for w, m in zip(ws, ms)]
    p = jnp.concatenate(tops, axis=1)

    t = None
    for _ in range(k_keep):
        t = jnp.max(p, axis=1, keepdims=True)
        p = jnp.where(p >= t, neg, p)

    km1 = jnp.float32(k_keep - 1)
    cnt = jnp.sum((s > t).astype(jnp.float32), axis=1, keepdims=True)

    def _cond(carry):
        _, c = carry
        return jnp.any(c > km1)

    def _body(carry):
        u, c = carry
        mn = jnp.min(jnp.where(s > u, s, jnp.float32(jnp.inf)),
                     axis=1, keepdims=True)
        u = jnp.where(c > km1, mn, u)
        c = jnp.sum((s > u).astype(jnp.float32), axis=1, keepdims=True)
        return u, c

    t, _ = jax.lax.while_loop(_cond, _body, (t, cnt))
    out_ref[...] = jnp.where(s >= t, jnp.maximum(s, 0.0), 0.0)


@functools.partial(jax.jit, static_argnames=("interpret",))
def kernel(x, w0, w1, interpret=False):
    n, d = x.shape
    rs = 200 if n % 200 == 0 else n

    h = jax.nn.relu(x * w0) * w1
    norm = jnp.linalg.norm(h, axis=-1, keepdims=True)
    h = (h / jnp.clip(norm, 1e-12, None)).astype(jnp.bfloat16)

    out = pl.pallas_call(
        functools.partial(_sim_topk_kernel, k_keep=K_KEEP),
        grid=(n // rs,),
        in_specs=[
            pl.BlockSpec((rs, d), lambda i: (i, 0)),
            pl.BlockSpec((n, d), lambda i: (0, 0)),
        ],
        out_specs=pl.BlockSpec((rs, n), lambda i: (i, 0)),
        out_shape=jax.ShapeDtypeStruct((n, n), jnp.float32),
        interpret=interpret,
    )(h, h)
    return out
